# baseline (speedup 1.0000x reference)
import functools

import jax
import jax.numpy as jnp
from jax import lax
from jax.experimental import pallas as pl
from jax.experimental.pallas import tpu as pltpu
from jax.experimental.pallas import tpu_sc as plsc

SEQ = 200
D = 64
SCALE = 8.0
LANES = 16
CHUNKS = ((0, 104), (104, 96))
NBUF = 3


@functools.lru_cache(maxsize=None)
def _make_prep(vocab: int):
    blk = 128

    def prep(t_ref, o_ref):
        o_ref[:, 0:D] = t_ref[...].T
        o_ref[:, D : 2 * D] = jnp.zeros((blk, D), jnp.float32)

    grid = (vocab + blk - 1) // blk
    return pl.pallas_call(
        prep,
        grid=(grid,),
        in_specs=[pl.BlockSpec((D, blk), lambda i: (0, i))],
        out_specs=pl.BlockSpec((blk, 2 * D), lambda i: (i, 0)),
        out_shape=jax.ShapeDtypeStruct((vocab, 2 * D), jnp.float32),
    )


@functools.lru_cache(maxsize=None)
def _make_kernel(batch: int, vocab: int):
    info = plsc.get_sparse_core_info()
    nc, ns = info.num_cores, info.num_subcores
    nw = nc * ns
    assert batch % nw == 0
    s_per_w = batch // nw

    mesh = plsc.VectorSubcoreMesh(core_axis_name="c", subcore_axis_name="s")

    @functools.partial(
        pl.kernel,
        mesh=mesh,
        out_type=jax.ShapeDtypeStruct((batch, SEQ, 2 * D), jnp.float32),
        scratch_types=[
            pltpu.VMEM((s_per_w * SEQ,), jnp.int32),
            pltpu.VMEM((NBUF, SEQ, 2 * D), jnp.float32),
            pltpu.VMEM((SEQ, 2 * D), jnp.float32),
            pltpu.SemaphoreType.DMA,
            pltpu.SemaphoreType.DMA,
        ],
    )
    def k(in_hbm, tok_hbm, pos_hbm, out_hbm, idx_v, rows_v, pos_v, gsem, wsem):
        wid = lax.axis_index("s") * nc + lax.axis_index("c")
        base = pl.multiple_of(wid * s_per_w, 8)
        ibase = pl.multiple_of(wid * (s_per_w * SEQ), 8)
        pltpu.sync_copy(pos_hbm, pos_v)
        pltpu.sync_copy(in_hbm.at[pl.ds(ibase, s_per_w * SEQ)], idx_v)

        def fire_gather(s, buf):
            for c0, clen in CHUNKS:
                pltpu.async_copy(
                    tok_hbm.at[idx_v.at[pl.ds(pl.multiple_of(s * SEQ + c0, 8), clen)]],
                    rows_v.at[buf, pl.ds(c0, clen)],
                    gsem,
                )

        def wait_gather(s, buf):
            for c0, clen in CHUNKS:
                pltpu.make_async_copy(
                    tok_hbm.at[idx_v.at[pl.ds(pl.multiple_of(s * SEQ + c0, 8), clen)]],
                    rows_v.at[buf, pl.ds(c0, clen)],
                    gsem,
                ).wait()

        def wait_write_one():
            pltpu.make_async_copy(rows_v.at[0], out_hbm.at[base], wsem).wait()

        fire_gather(0, 0)

        def body(s, carry):
            buf = lax.rem(s, NBUF)
            nxt = lax.rem(s + 1, NBUF)

            @pl.when(s >= NBUF - 1)
            def _():
                wait_write_one()

            @pl.when(s + 1 < s_per_w)
            def _():
                fire_gather(s + 1, nxt)

            wait_gather(s, buf)

            @plsc.parallel_loop(0, SEQ, step=1, unroll=8)
            def _(r):
                for j in range(D // LANES):
                    sl = pl.ds(j * LANES, LANES)
                    rows_v[buf, r, sl] = rows_v[buf, r, sl] * SCALE + pos_v[r, sl]

            pltpu.async_copy(rows_v.at[buf], out_hbm.at[base + s], wsem)
            return carry

        lax.fori_loop(0, s_per_w, body, 0)
        for _ in range(NBUF - 1):
            wait_write_one()

    return k


def kernel(input_, token_table, position_table):
    batch, seq = input_.shape
    vocab, d = token_table.shape
    assert seq == SEQ and d == D
    tok2 = _make_prep(vocab)(token_table.T)
    pos2 = jnp.concatenate([position_table, position_table], axis=1)
    idxf = input_.astype(jnp.int32).reshape(-1)
    k = _make_kernel(batch, vocab)
    out = k(idxf, tok2, pos2)
    return out[:, :, :D]


# speedup vs baseline: 3.3586x; 3.3586x over previous
import functools

import jax
import jax.numpy as jnp
from jax import lax
from jax.experimental import pallas as pl
from jax.experimental.pallas import tpu as pltpu
from jax.experimental.pallas import tpu_sc as plsc

SEQ = 200
D = 64
SCALE = 8.0
LANES = 16
CHUNKS = ((0, 104), (104, 96))
NBUF = 3


@functools.lru_cache(maxsize=None)
def _make_prep(vocab: int):
    blk = 1024

    def prep(t_ref, o_ref):
        eye = jax.lax.broadcasted_iota(jnp.int32, (D, D), 0) == jax.lax.broadcasted_iota(jnp.int32, (D, D), 1)
        eyef = eye.astype(jnp.float32)
        o_ref[:, 0:D] = jax.lax.dot_general(
            t_ref[...], eyef, (((0,), (0,)), ((), ())),
            preferred_element_type=jnp.float32,
            precision=jax.lax.Precision.HIGHEST,
        )
        o_ref[:, D : 2 * D] = jnp.zeros((blk, D), jnp.float32)

    grid = (vocab + blk - 1) // blk
    return pl.pallas_call(
        prep,
        grid=(grid,),
        in_specs=[pl.BlockSpec((D, blk), lambda i: (0, i))],
        out_specs=pl.BlockSpec((blk, 2 * D), lambda i: (i, 0)),
        out_shape=jax.ShapeDtypeStruct((vocab, 2 * D), jnp.float32),
    )


@functools.lru_cache(maxsize=None)
def _make_kernel(batch: int, vocab: int):
    info = plsc.get_sparse_core_info()
    nc, ns = info.num_cores, info.num_subcores
    nw = nc * ns
    assert batch % nw == 0
    s_per_w = batch // nw

    mesh = plsc.VectorSubcoreMesh(core_axis_name="c", subcore_axis_name="s")

    @functools.partial(
        pl.kernel,
        mesh=mesh,
        out_type=jax.ShapeDtypeStruct((batch, SEQ, 2 * D), jnp.float32),
        scratch_types=[
            pltpu.VMEM((s_per_w * SEQ,), jnp.int32),
            pltpu.VMEM((NBUF, SEQ, 2 * D), jnp.float32),
            pltpu.VMEM((SEQ, 2 * D), jnp.float32),
            pltpu.SemaphoreType.DMA,
            pltpu.SemaphoreType.DMA,
        ],
    )
    def k(in_hbm, tok_hbm, pos_hbm, out_hbm, idx_v, rows_v, pos_v, gsem, wsem):
        wid = lax.axis_index("s") * nc + lax.axis_index("c")
        base = pl.multiple_of(wid * s_per_w, 8)
        ibase = pl.multiple_of(wid * (s_per_w * SEQ), 8)
        pltpu.sync_copy(pos_hbm, pos_v)
        pltpu.sync_copy(in_hbm.at[pl.ds(ibase, s_per_w * SEQ)], idx_v)

        def fire_gather(s, buf):
            for c0, clen in CHUNKS:
                pltpu.async_copy(
                    tok_hbm.at[idx_v.at[pl.ds(pl.multiple_of(s * SEQ + c0, 8), clen)]],
                    rows_v.at[buf, pl.ds(c0, clen)],
                    gsem,
                )

        def wait_gather(s, buf):
            for c0, clen in CHUNKS:
                pltpu.make_async_copy(
                    tok_hbm.at[idx_v.at[pl.ds(pl.multiple_of(s * SEQ + c0, 8), clen)]],
                    rows_v.at[buf, pl.ds(c0, clen)],
                    gsem,
                ).wait()

        def wait_write_one():
            pltpu.make_async_copy(rows_v.at[0], out_hbm.at[base], wsem).wait()

        fire_gather(0, 0)

        def body(s, carry):
            buf = lax.rem(s, NBUF)
            nxt = lax.rem(s + 1, NBUF)

            @pl.when(s >= NBUF - 1)
            def _():
                wait_write_one()

            @pl.when(s + 1 < s_per_w)
            def _():
                fire_gather(s + 1, nxt)

            wait_gather(s, buf)

            @plsc.parallel_loop(0, SEQ, step=1, unroll=8)
            def _(r):
                for j in range(D // LANES):
                    sl = pl.ds(j * LANES, LANES)
                    rows_v[buf, r, sl] = rows_v[buf, r, sl] * SCALE + pos_v[r, sl]

            pltpu.async_copy(rows_v.at[buf], out_hbm.at[base + s], wsem)
            return carry

        lax.fori_loop(0, s_per_w, body, 0)
        for _ in range(NBUF - 1):
            wait_write_one()

    return k


def kernel(input_, token_table, position_table):
    batch, seq = input_.shape
    vocab, d = token_table.shape
    assert seq == SEQ and d == D
    tok2 = _make_prep(vocab)(token_table.T)
    pos2 = jnp.concatenate([position_table, position_table], axis=1)
    idxf = input_.astype(jnp.int32).reshape(-1)
    k = _make_kernel(batch, vocab)
    out = k(idxf, tok2, pos2)
    return out[:, :, :D]


# speedup vs baseline: 4.5163x; 1.3447x over previous
import functools

import jax
import jax.numpy as jnp
from jax import lax
from jax.experimental import pallas as pl
from jax.experimental.pallas import tpu as pltpu
from jax.experimental.pallas import tpu_sc as plsc

SEQ = 200
D = 64
SCALE = 8.0
LANES = 16
CHUNKS = ((0, 104), (104, 96))
NBUF = 3


@functools.lru_cache(maxsize=None)
def _make_prep(vocab: int):
    blk = 1024

    def prep(t_ref, o_ref):
        eye = jax.lax.broadcasted_iota(jnp.int32, (D, D), 0) == jax.lax.broadcasted_iota(jnp.int32, (D, D), 1)
        eyef = eye.astype(jnp.float32)
        o_ref[:, 0:D] = jax.lax.dot_general(
            t_ref[...], eyef, (((0,), (0,)), ((), ())),
            preferred_element_type=jnp.float32,
            precision=jax.lax.Precision.HIGHEST,
        )
        o_ref[:, D : 2 * D] = jnp.zeros((blk, D), jnp.float32)

    grid = (vocab + blk - 1) // blk
    return pl.pallas_call(
        prep,
        grid=(grid,),
        in_specs=[pl.BlockSpec((D, blk), lambda i: (0, i))],
        out_specs=pl.BlockSpec((blk, 2 * D), lambda i: (i, 0)),
        out_shape=jax.ShapeDtypeStruct((vocab, 2 * D), jnp.float32),
    )


@functools.lru_cache(maxsize=None)
def _make_kernel(batch: int, vocab: int):
    info = plsc.get_sparse_core_info()
    nc, ns = info.num_cores, info.num_subcores
    nw = nc * ns
    assert batch % nw == 0
    s_per_w = batch // nw

    mesh = plsc.VectorSubcoreMesh(core_axis_name="c", subcore_axis_name="s")

    @functools.partial(
        pl.kernel,
        mesh=mesh,
        out_type=jax.ShapeDtypeStruct((batch, SEQ, 2 * D), jnp.float32),
        scratch_types=[
            pltpu.VMEM((s_per_w * SEQ,), jnp.int32),
            pltpu.VMEM((NBUF, SEQ, 2 * D), jnp.float32),
            pltpu.VMEM((SEQ, 2 * D), jnp.float32),
            pltpu.SemaphoreType.DMA,
            pltpu.SemaphoreType.DMA,
        ],
    )
    def k(in_hbm, tok_hbm, pos_hbm, out_hbm, idx_v, rows_v, pos_v, gsem, wsem):
        wid = lax.axis_index("s") * nc + lax.axis_index("c")
        base = pl.multiple_of(wid * s_per_w, 8)
        ibase = pl.multiple_of(wid * (s_per_w * SEQ), 8)
        pltpu.sync_copy(pos_hbm, pos_v)
        pltpu.sync_copy(in_hbm.at[pl.ds(ibase, s_per_w * SEQ)], idx_v)

        def fire_gather(s, buf):
            for c0, clen in CHUNKS:
                pltpu.async_copy(
                    tok_hbm.at[idx_v.at[pl.ds(pl.multiple_of(s * SEQ + c0, 8), clen)]],
                    rows_v.at[buf, pl.ds(c0, clen)],
                    gsem,
                )

        def wait_gather(s, buf):
            for c0, clen in CHUNKS:
                pltpu.make_async_copy(
                    tok_hbm.at[idx_v.at[pl.ds(pl.multiple_of(s * SEQ + c0, 8), clen)]],
                    rows_v.at[buf, pl.ds(c0, clen)],
                    gsem,
                ).wait()

        def wait_write_one():
            pltpu.make_async_copy(rows_v.at[0], out_hbm.at[base], wsem).wait()

        fire_gather(0, 0)

        def body(s, carry):
            buf = lax.rem(s, NBUF)
            nxt = lax.rem(s + 1, NBUF)

            @pl.when(s >= NBUF - 1)
            def _():
                wait_write_one()

            @pl.when(s + 1 < s_per_w)
            def _():
                fire_gather(s + 1, nxt)

            wait_gather(s, buf)

            @plsc.parallel_loop(0, SEQ, step=1, unroll=8)
            def _(r):
                for j in range(D // LANES):
                    sl = pl.ds(j * LANES, LANES)
                    rows_v[buf, r, sl] = rows_v[buf, r, sl] * SCALE + pos_v[r, sl]

            pltpu.async_copy(rows_v.at[buf], out_hbm.at[base + s], wsem)
            return carry

        lax.fori_loop(0, s_per_w, body, 0)
        for _ in range(NBUF - 1):
            wait_write_one()

    return k


def kernel(input_, token_table, position_table):
    batch, seq = input_.shape
    vocab, d = token_table.shape
    assert seq == SEQ and d == D
    tok2 = jnp.pad(token_table, ((0, 0), (0, D)))
    pos2 = jnp.concatenate([position_table, position_table], axis=1)
    idxf = input_.astype(jnp.int32).reshape(-1)
    k = _make_kernel(batch, vocab)
    out = k(idxf, tok2, pos2)
    return out[:, :, :D]


# speedup vs baseline: 4.5319x; 1.0035x over previous
import functools

import jax
import jax.numpy as jnp
from jax import lax
from jax.experimental import pallas as pl
from jax.experimental.pallas import tpu as pltpu
from jax.experimental.pallas import tpu_sc as plsc

SEQ = 200
D = 64
SCALE = 8.0
LANES = 16
CHUNKS = ((0, 104), (104, 96))
NBUF = 3


@functools.lru_cache(maxsize=None)
def _make_kernel(batch: int, vocab: int):
    info = plsc.get_sparse_core_info()
    nc, ns = info.num_cores, info.num_subcores
    nw = nc * ns
    assert batch % nw == 0
    s_per_w = batch // nw

    mesh = plsc.VectorSubcoreMesh(core_axis_name="c", subcore_axis_name="s")

    @functools.partial(
        pl.kernel,
        mesh=mesh,
        out_type=jax.ShapeDtypeStruct((batch, SEQ, 2 * D), jnp.float32),
        scratch_types=[
            pltpu.VMEM((s_per_w * SEQ,), jnp.int32),
            pltpu.VMEM((NBUF, SEQ, 2 * D), jnp.float32),
            pltpu.VMEM((SEQ, 2 * D), jnp.float32),
            pltpu.SemaphoreType.DMA,
            pltpu.SemaphoreType.DMA,
        ],
    )
    def k(in_hbm, tok_hbm, pos_hbm, out_hbm, idx_v, rows_v, pos_v, gsem, wsem):
        wid = lax.axis_index("s") * nc + lax.axis_index("c")
        base = pl.multiple_of(wid * s_per_w, 8)
        ibase = pl.multiple_of(wid * (s_per_w * SEQ), 8)
        pltpu.sync_copy(pos_hbm, pos_v)
        pltpu.sync_copy(in_hbm.at[pl.ds(ibase, s_per_w * SEQ)], idx_v)

        def fire_gather(s, buf):
            for c0, clen in CHUNKS:
                pltpu.async_copy(
                    tok_hbm.at[idx_v.at[pl.ds(pl.multiple_of(s * SEQ + c0, 8), clen)]],
                    rows_v.at[buf, pl.ds(c0, clen)],
                    gsem,
                )

        def wait_gather(s, buf):
            for c0, clen in CHUNKS:
                pltpu.make_async_copy(
                    tok_hbm.at[idx_v.at[pl.ds(pl.multiple_of(s * SEQ + c0, 8), clen)]],
                    rows_v.at[buf, pl.ds(c0, clen)],
                    gsem,
                ).wait()

        def wait_write_one():
            pltpu.make_async_copy(rows_v.at[0], out_hbm.at[base], wsem).wait()

        fire_gather(0, 0)

        def body(s, carry):
            buf = lax.rem(s, NBUF)
            nxt = lax.rem(s + 1, NBUF)

            @pl.when(s >= NBUF - 1)
            def _():
                wait_write_one()

            @pl.when(s + 1 < s_per_w)
            def _():
                fire_gather(s + 1, nxt)

            wait_gather(s, buf)

            @plsc.parallel_loop(0, SEQ, step=1, unroll=8)
            def _(r):
                for j in range(D // LANES):
                    sl = pl.ds(j * LANES, LANES)
                    rows_v[buf, r, sl] = rows_v[buf, r, sl] * SCALE + pos_v[r, sl]

            pltpu.async_copy(rows_v.at[buf], out_hbm.at[base + s], wsem)
            return carry

        lax.fori_loop(0, s_per_w, body, 0)
        for _ in range(NBUF - 1):
            wait_write_one()

    return k


def kernel(input_, token_table, position_table):
    batch, seq = input_.shape
    vocab, d = token_table.shape
    assert seq == SEQ and d == D
    tok2 = jnp.pad(token_table, ((0, 0), (0, D)))
    pos2 = jnp.concatenate([position_table, position_table], axis=1)
    idxf = input_.astype(jnp.int32).reshape(-1)
    k = _make_kernel(batch, vocab)
    out = k(idxf, tok2, pos2)
    return out[:, :, :D]
